# baseline (speedup 1.0000x reference)
import jax
import jax.numpy as jnp
from jax import lax
from jax.experimental import pallas as pl
from jax.experimental.pallas import tpu as pltpu
from jax.experimental.pallas import tpu_sc as plsc

NC, NS, L = 2, 16, 16
NW = NC * NS
B = 16384
D = 64
BPW = B // NW
CH = 128
NCH = BPW // CH


def _body(hidx_hbm, ridx_hbm, tidx_hbm, ent_hbm, rel_hbm, out_hbm,
          hidx_v, ridx_v, tidx_v, h2_v, r2_v, t2_v,
          hrows, rrows, trows, score_v, sem_h, sem_r, sem_t):
    wid = lax.axis_index("s") * NC + lax.axis_index("c")
    pltpu.sync_copy(hidx_hbm.at[wid], hidx_v)
    pltpu.sync_copy(ridx_hbm.at[wid], ridx_v)
    pltpu.sync_copy(tidx_hbm.at[wid], tidx_v)
    lane = lax.iota(jnp.int32, L)
    for j in range(NCH):
        def halve(k, _):
            h2_v[pl.ds(k * L, L)] = hidx_v[j, pl.ds(k * L, L)] >> 1
            r2_v[pl.ds(k * L, L)] = ridx_v[j, pl.ds(k * L, L)] >> 1
            t2_v[pl.ds(k * L, L)] = tidx_v[j, pl.ds(k * L, L)] >> 1
            return 0

        lax.fori_loop(0, CH // L, halve, 0)
        cph = pltpu.async_copy(ent_hbm.at[h2_v], hrows, sem_h)
        cpr = pltpu.async_copy(rel_hbm.at[r2_v], rrows, sem_r)
        cpt = pltpu.async_copy(ent_hbm.at[t2_v], trows, sem_t)
        cph.wait()
        cpr.wait()
        cpt.wait()

        def compute(g, carry):
            rows = g * L + lane
            hcol = (hidx_v[j, pl.ds(g * L, L)] & 1) * D
            rcol = (ridx_v[j, pl.ds(g * L, L)] & 1) * D
            tcol = (tidx_v[j, pl.ds(g * L, L)] & 1) * D
            acc = jnp.zeros((L,), jnp.float32)
            for d in range(D):
                hv = plsc.load_gather(hrows, [rows, hcol + d])
                rv = plsc.load_gather(rrows, [rows, rcol + d])
                tv = plsc.load_gather(trows, [rows, tcol + d])
                acc = acc + jnp.abs(hv + rv - tv)
            score_v[pl.ds(g * L, L)] = acc
            return carry

        lax.fori_loop(0, CH // L, compute, 0)
        base = wid * BPW + j * CH
        pltpu.sync_copy(score_v, out_hbm.at[pl.ds(base, CH)])


def kernel(sample, entity_embedding, relation_embedding):
    hidx = sample[:, 0].reshape(NW, NCH, CH)
    ridx = sample[:, 1].reshape(NW, NCH, CH)
    tidx = sample[:, 2].reshape(NW, NCH, CH)
    ent2 = entity_embedding.reshape(NENT := entity_embedding.shape[0] // 2, 2 * D)
    rel2 = relation_embedding.reshape(relation_embedding.shape[0] // 2, 2 * D)
    del NENT
    mesh = plsc.VectorSubcoreMesh(
        core_axis_name="c", subcore_axis_name="s",
        num_cores=NC, num_subcores=NS)
    f = pl.kernel(
        _body,
        out_type=jax.ShapeDtypeStruct((B,), jnp.float32),
        mesh=mesh,
        compiler_params=pltpu.CompilerParams(
            needs_layout_passes=False, use_tc_tiling_on_sc=True),
        scratch_types=[
            pltpu.VMEM((NCH, CH), jnp.int32),
            pltpu.VMEM((NCH, CH), jnp.int32),
            pltpu.VMEM((NCH, CH), jnp.int32),
            pltpu.VMEM((CH,), jnp.int32),
            pltpu.VMEM((CH,), jnp.int32),
            pltpu.VMEM((CH,), jnp.int32),
            pltpu.VMEM((CH, 2 * D), jnp.float32),
            pltpu.VMEM((CH, 2 * D), jnp.float32),
            pltpu.VMEM((CH, 2 * D), jnp.float32),
            pltpu.VMEM((CH,), jnp.float32),
            pltpu.SemaphoreType.DMA,
            pltpu.SemaphoreType.DMA,
            pltpu.SemaphoreType.DMA,
        ],
    )
    score = f(hidx, ridx, tidx, ent2, rel2)
    return score.reshape(B, 1)
